# baseline (speedup 1.0000x reference)
import functools

import jax
import jax.numpy as jnp
from jax import lax
from jax.experimental import pallas as pl
from jax.experimental.pallas import tpu as pltpu
from jax.experimental.pallas import tpu_sc as plsc


def _build_gather(n, d, n_workers, chunk):
    n_per_w = n // n_workers
    n_chunks = n_per_w // chunk
    mesh = plsc.VectorSubcoreMesh(core_axis_name="c", subcore_axis_name="s")
    nc = 2

    @functools.partial(
        pl.kernel,
        mesh=mesh,
        out_type=jax.ShapeDtypeStruct((n, d), jnp.float32),
        scratch_types=[
            pltpu.VMEM((chunk,), jnp.int32),
            pltpu.VMEM((chunk, d), jnp.float32),
            pltpu.SemaphoreType.DMA,
        ],
        compiler_params=pltpu.CompilerParams(use_tc_tiling_on_sc=False),
    )
    def gather_kernel(table_hbm, idx_hbm, out_hbm, idx_v, rows_v, sem):
        wid = lax.axis_index("s") * nc + lax.axis_index("c")
        base = wid * n_per_w

        def body(c, _):
            off = base + c * chunk
            pltpu.sync_copy(idx_hbm.at[pl.ds(off, chunk)], idx_v)
            pltpu.async_copy(table_hbm.at[idx_v], rows_v, sem).wait()
            pltpu.sync_copy(rows_v, out_hbm.at[pl.ds(off, chunk)])
            return ()

        lax.fori_loop(0, n_chunks, body, (), unroll=False)

    return gather_kernel


def kernel(x, table):
    b, l = x.shape
    v, d = table.shape
    n = b * l
    idx = x.reshape(n).astype(jnp.int32)
    out = _build_gather(n, d, 32, 1024)(table, idx)
    return out.reshape(b, l, d)


# speedup vs baseline: 1.0067x; 1.0067x over previous
import functools

import jax
import jax.numpy as jnp
from jax import lax
from jax.experimental import pallas as pl
from jax.experimental.pallas import tpu as pltpu
from jax.experimental.pallas import tpu_sc as plsc


def _build_gather(n, d, n_workers, chunk):
    n_per_w = n // n_workers
    n_chunks = n_per_w // chunk
    mesh = plsc.VectorSubcoreMesh(core_axis_name="c", subcore_axis_name="s")
    nc = 2

    @functools.partial(
        pl.kernel,
        mesh=mesh,
        out_type=jax.ShapeDtypeStruct((n, d), jnp.float32),
        scratch_types=[
            pltpu.VMEM((chunk,), jnp.int32),
            pltpu.VMEM((chunk,), jnp.int32),
            pltpu.VMEM((chunk, d), jnp.float32),
            pltpu.VMEM((chunk, d), jnp.float32),
            pltpu.SemaphoreType.DMA,
            pltpu.SemaphoreType.DMA,
            pltpu.SemaphoreType.DMA,
            pltpu.SemaphoreType.DMA,
            pltpu.SemaphoreType.DMA,
            pltpu.SemaphoreType.DMA,
        ],
        compiler_params=pltpu.CompilerParams(use_tc_tiling_on_sc=False),
    )
    def gather_kernel(table_hbm, idx_hbm, out_hbm,
                      idx0, idx1, rows0, rows1, si0, si1, sg0, sg1, ss0, ss1):
        wid = lax.axis_index("s") * nc + lax.axis_index("c")
        base = wid * n_per_w
        idx_b = [idx0, idx1]
        rows_b = [rows0, rows1]
        si = [si0, si1]
        sg = [sg0, sg1]
        ss = [ss0, ss1]

        def off(c):
            return base + c * chunk

        h_idx = [
            pltpu.async_copy(idx_hbm.at[pl.ds(off(0), chunk)], idx_b[0], si[0]),
            pltpu.async_copy(idx_hbm.at[pl.ds(off(1), chunk)], idx_b[1], si[1]),
        ]
        h_s = [None, None]
        for c in range(n_chunks):
            p = c % 2
            if c >= 2:
                h_s[p].wait()
            h_idx[p].wait()
            h_g = pltpu.async_copy(table_hbm.at[idx_b[p]], rows_b[p], sg[p])
            h_g.wait()
            if c + 2 < n_chunks:
                h_idx[p] = pltpu.async_copy(
                    idx_hbm.at[pl.ds(off(c + 2), chunk)], idx_b[p], si[p])
            h_s[p] = pltpu.async_copy(
                rows_b[p], out_hbm.at[pl.ds(off(c), chunk)], ss[p])
        h_s[0].wait()
        h_s[1].wait()

    return gather_kernel


def kernel(x, table):
    b, l = x.shape
    v, d = table.shape
    n = b * l
    idx = x.reshape(n).astype(jnp.int32)
    out = _build_gather(n, d, 32, 512)(table, idx)
    return out.reshape(b, l, d)
